# baseline (speedup 1.0000x reference)
import functools

import jax
import jax.numpy as jnp
import numpy as np
from jax import lax
from jax.experimental import pallas as pl
from jax.experimental.pallas import tpu as pltpu
from jax.experimental.pallas import tpu_sc as plsc

BATCH = 4096
N_ENT = 50
N_PRED = 20
D = 64
OUT_D = 128
E_VOCAB = 1000000
P_VOCAB = 100000
DP = 128

NC = 2
NS = 16
NW = NC * NS
ROWS_W = BATCH // NW

E_CH_ROWS = 2
E_CH_REAL = E_CH_ROWS * N_ENT
E_CH = 104
E_CHUNKS_W = ROWS_W // E_CH_ROWS
P_CH_ROWS = 4
P_CH = P_CH_ROWS * N_PRED
P_CHUNKS_W = ROWS_W // P_CH_ROWS

E_SCALE = 1.0 / (2.0 * N_ENT)
P_SCALE = 1.0 / (2.0 * N_PRED)

_mesh = plsc.VectorSubcoreMesh(core_axis_name="c", subcore_axis_name="s")


_BN = 4096


def _relayout_body(lo_ref, hi_ref, out_ref):
    lo = lo_ref[...]
    hi = hi_ref[...]
    out_ref[...] = jnp.concatenate([lo.T, hi.T], axis=1)


def _make_relayout(vocab, half_blocks):
    n_blocks = (vocab + _BN - 1) // _BN
    last = n_blocks - 1
    return pl.pallas_call(
        _relayout_body,
        grid=(half_blocks,),
        in_specs=[
            pl.BlockSpec((D, _BN), lambda i: (0, i)),
            pl.BlockSpec((D, _BN),
                         lambda i: (0, jnp.minimum(i + half_blocks, last))),
        ],
        out_specs=pl.BlockSpec((_BN, DP), lambda i: (i, 0)),
        out_shape=jax.ShapeDtypeStruct((half_blocks * _BN, DP), jnp.float32),
    )


E_HALF_BLOCKS = 123
P_HALF_BLOCKS = 13
E_HALF = E_HALF_BLOCKS * _BN
P_HALF = P_HALF_BLOCKS * _BN

_relayout_e = _make_relayout(E_VOCAB, E_HALF_BLOCKS)
_relayout_p = _make_relayout(P_VOCAB, P_HALF_BLOCKS)


E_GROUPS = [(g * 16, 0, 16) for g in range(6)] + [(88, 8, 12)]
P_GROUPS = [(g * 16, 0, 16) for g in range(5)]


def _make_gather_pool(n_chunks_w, ch, groups, n_per_row, ch_rows, scale):
    @functools.partial(
        pl.kernel,
        out_type=jax.ShapeDtypeStruct((BATCH, D), jnp.float32),
        mesh=_mesh,
        scratch_types=[
            pltpu.VMEM((n_chunks_w, ch), jnp.int32),
            pltpu.VMEM((n_chunks_w, ch), jnp.int32),
            pltpu.VMEM((ch, DP), jnp.float32),
            pltpu.VMEM((ch, DP), jnp.float32),
            pltpu.VMEM((ch, DP), jnp.float32),
            pltpu.VMEM((ch, DP), jnp.float32),
            pltpu.VMEM((ROWS_W, D), jnp.float32),
            pltpu.SemaphoreType.DMA,
            pltpu.SemaphoreType.DMA,
            pltpu.SemaphoreType.DMA,
            pltpu.SemaphoreType.DMA,
        ],
    )
    def _gather_pool(idx_hbm, off_hbm, tab_hbm, out_hbm,
                     idx_v, off_v, buf0, buf1, buf2, buf3, sem_v,
                     dsem0, dsem1, dsem2, dsem3):
        wid = lax.axis_index("s") * NC + lax.axis_index("c")

        pltpu.sync_copy(idx_hbm.at[pl.ds(wid * n_chunks_w, n_chunks_w)], idx_v)
        pltpu.sync_copy(off_hbm.at[pl.ds(wid * n_chunks_w, n_chunks_w)], off_v)

        def start(k, buf, sem):
            pltpu.async_copy(tab_hbm.at[idx_v.at[k]], buf, sem)

        def wait(buf, sem):
            pltpu.make_async_copy(tab_hbm.at[pl.ds(0, ch)], buf, sem).wait()

        zeros4 = tuple(jnp.zeros((16,), jnp.float32) for _ in range(4))

        def reduce(buf, k):
            accs = [list(zeros4) for _ in range(ch_rows)]
            for base, j0, j1 in groups:
                offs = off_v[k, pl.ds(base, 16)]
                for j in range(j0, j1):
                    b = base + j
                    acc = accs[b // n_per_row]
                    off = offs[j]
                    for c in range(4):
                        acc[c] = acc[c] + buf[b, pl.ds(off + 16 * c, 16)]
            for r in range(ch_rows):
                for c in range(4):
                    sem_v[k * ch_rows + r, pl.ds(16 * c, 16)] = (
                        accs[r][c] * scale)

        bufs = (buf0, buf1, buf2, buf3)
        sems = (dsem0, dsem1, dsem2, dsem3)
        for t in range(1, 4):
            start(t - 1, bufs[t], sems[t])

        def loop(j, carry):
            k0 = 4 * j
            for t in range(4):
                start(jnp.minimum(k0 + t + 3, n_chunks_w - 1),
                      bufs[t], sems[t])
                wait(bufs[(t + 1) % 4], sems[(t + 1) % 4])
                reduce(bufs[(t + 1) % 4], k0 + t)
            return carry

        lax.fori_loop(0, n_chunks_w // 4, loop, 0)
        for t in range(1, 4):
            wait(bufs[t], sems[t])

        pltpu.sync_copy(sem_v, out_hbm.at[pl.ds(wid * ROWS_W, ROWS_W)])

    return _gather_pool


_ent_sc = _make_gather_pool(E_CHUNKS_W, E_CH, E_GROUPS, N_ENT, E_CH_ROWS,
                            E_SCALE)
_pred_sc = _make_gather_pool(P_CHUNKS_W, P_CH, P_GROUPS, N_PRED, P_CH_ROWS,
                             P_SCALE)


def _mlp_tc_body(seme_ref, semp_ref, mot_ref, w1_ref, b1_ref, w2_ref, b2_ref,
                 wft_ref, wfb_ref, bf_ref, out_ref):
    x = mot_ref[...]
    h = jnp.maximum(
        jnp.dot(x, w1_ref[...], preferred_element_type=jnp.float32)
        + b1_ref[...], 0.0)
    m = jnp.maximum(
        jnp.dot(h, w2_ref[...], preferred_element_type=jnp.float32)
        + b2_ref[...], 0.0)
    s = seme_ref[...] + semp_ref[...]
    f = (jnp.dot(s, wft_ref[...], preferred_element_type=jnp.float32)
         + jnp.dot(m, wfb_ref[...], preferred_element_type=jnp.float32)
         + bf_ref[...])
    out_ref[...] = jnp.maximum(f, 0.0)


_BM = 1024

_mlp_tc = pl.pallas_call(
    _mlp_tc_body,
    grid=(BATCH // _BM,),
    in_specs=[
        pl.BlockSpec((_BM, D), lambda i: (i, 0)),
        pl.BlockSpec((_BM, D), lambda i: (i, 0)),
        pl.BlockSpec((_BM, D), lambda i: (i, 0)),
        pl.BlockSpec((D, D), lambda i: (0, 0)),
        pl.BlockSpec((1, D), lambda i: (0, 0)),
        pl.BlockSpec((D, D), lambda i: (0, 0)),
        pl.BlockSpec((1, D), lambda i: (0, 0)),
        pl.BlockSpec((D, OUT_D), lambda i: (0, 0)),
        pl.BlockSpec((D, OUT_D), lambda i: (0, 0)),
        pl.BlockSpec((1, OUT_D), lambda i: (0, 0)),
    ],
    out_specs=pl.BlockSpec((_BM, OUT_D), lambda i: (i, 0)),
    out_shape=jax.ShapeDtypeStruct((BATCH, OUT_D), jnp.float32),
)


def kernel(entity_indices, predicate_indices, motif_counts_batch,
           entity_table, predicate_table, W1, b1, W2, b2, Wf, bf):
    eidx = entity_indices.astype(jnp.int32).reshape(
        BATCH * N_ENT // E_CH_REAL, E_CH_REAL)
    eidx = jnp.pad(eidx, ((0, 0), (0, E_CH - E_CH_REAL)), mode="edge")
    pidx = predicate_indices.astype(jnp.int32).reshape(
        BATCH * N_PRED // P_CH, P_CH)
    e_hi = eidx >= E_HALF
    p_hi = pidx >= P_HALF
    eoff = e_hi.astype(jnp.int32) * D
    poff = p_hi.astype(jnp.int32) * D
    eidx2 = eidx - e_hi.astype(jnp.int32) * E_HALF
    pidx2 = pidx - p_hi.astype(jnp.int32) * P_HALF
    ptab = _relayout_p(predicate_table.astype(jnp.float32).T,
                       predicate_table.astype(jnp.float32).T)
    sem_p = _pred_sc(pidx2, poff, ptab)
    etab = _relayout_e(entity_table.astype(jnp.float32).T,
                       entity_table.astype(jnp.float32).T)
    sem_e = _ent_sc(eidx2, eoff, etab)
    return _mlp_tc(sem_e, sem_p, motif_counts_batch,
                   W1, b1.reshape(1, D), W2, b2.reshape(1, D),
                   Wf[:D], Wf[D:], bf.reshape(1, OUT_D))


# speedup vs baseline: 1.0178x; 1.0178x over previous
import functools

import jax
import jax.numpy as jnp
import numpy as np
from jax import lax
from jax.experimental import pallas as pl
from jax.experimental.pallas import tpu as pltpu
from jax.experimental.pallas import tpu_sc as plsc

BATCH = 4096
N_ENT = 50
N_PRED = 20
D = 64
OUT_D = 128
E_VOCAB = 1000000
P_VOCAB = 100000
DP = 128

NC = 2
NS = 16
NW = NC * NS
ROWS_W = BATCH // NW

E_CH_ROWS = 2
E_CH_REAL = E_CH_ROWS * N_ENT
E_CH = 104
E_CHUNKS_W = ROWS_W // E_CH_ROWS
P_CH_ROWS = 4
P_CH = P_CH_ROWS * N_PRED
P_CHUNKS_W = ROWS_W // P_CH_ROWS

E_SCALE = 1.0 / (2.0 * N_ENT)
P_SCALE = 1.0 / (2.0 * N_PRED)

_mesh = plsc.VectorSubcoreMesh(core_axis_name="c", subcore_axis_name="s")


_BN = 4096


def _relayout_body(lo_ref, hi_ref, out_ref):
    lo = lo_ref[...]
    hi = hi_ref[...]
    out_ref[...] = jnp.concatenate([lo.T, hi.T], axis=1)


def _make_relayout(vocab, half_blocks):
    n_blocks = (vocab + _BN - 1) // _BN
    last = n_blocks - 1
    return pl.pallas_call(
        _relayout_body,
        grid=(half_blocks,),
        in_specs=[
            pl.BlockSpec((D, _BN), lambda i: (0, i)),
            pl.BlockSpec((D, _BN),
                         lambda i: (0, jnp.minimum(i + half_blocks, last))),
        ],
        out_specs=pl.BlockSpec((_BN, DP), lambda i: (i, 0)),
        out_shape=jax.ShapeDtypeStruct((half_blocks * _BN, DP), jnp.float32),
    )


E_HALF_BLOCKS = 123
P_HALF_BLOCKS = 13
E_HALF = E_HALF_BLOCKS * _BN
P_HALF = P_HALF_BLOCKS * _BN

_relayout_e = _make_relayout(E_VOCAB, E_HALF_BLOCKS)
_relayout_p = _make_relayout(P_VOCAB, P_HALF_BLOCKS)


E_GROUPS = [(g * 16, 0, 16) for g in range(6)] + [(88, 8, 12)]
P_GROUPS = [(g * 16, 0, 16) for g in range(5)]


def _make_gather_pool(n_chunks_w, ch, groups, n_per_row, ch_rows, scale):
    @functools.partial(
        pl.kernel,
        out_type=jax.ShapeDtypeStruct((BATCH, D), jnp.float32),
        mesh=_mesh,
        scratch_types=[
            pltpu.VMEM((n_chunks_w, ch), jnp.int32),
            pltpu.VMEM((n_chunks_w, ch), jnp.int32),
            pltpu.VMEM((ch, DP), jnp.float32),
            pltpu.VMEM((ch, DP), jnp.float32),
            pltpu.VMEM((ROWS_W, D), jnp.float32),
            pltpu.SemaphoreType.DMA,
            pltpu.SemaphoreType.DMA,
        ],
    )
    def _gather_pool(idx_hbm, off_hbm, tab_hbm, out_hbm,
                     idx_v, off_v, buf0, buf1, sem_v, dsem0, dsem1):
        wid = lax.axis_index("s") * NC + lax.axis_index("c")

        pltpu.sync_copy(idx_hbm.at[pl.ds(wid * n_chunks_w, n_chunks_w)], idx_v)
        pltpu.sync_copy(off_hbm.at[pl.ds(wid * n_chunks_w, n_chunks_w)], off_v)

        def start(k, buf, sem):
            pltpu.async_copy(tab_hbm.at[idx_v.at[k]], buf, sem)

        def wait(buf, sem):
            pltpu.make_async_copy(tab_hbm.at[pl.ds(0, ch)], buf, sem).wait()

        zeros4 = tuple(jnp.zeros((16,), jnp.float32) for _ in range(4))

        def reduce(buf, k):
            accs = [list(zeros4) for _ in range(ch_rows)]
            for base, j0, j1 in groups:
                offs = off_v[k, pl.ds(base, 16)]
                for j in range(j0, j1):
                    b = base + j
                    acc = accs[b // n_per_row]
                    off = offs[j]
                    for c in range(4):
                        acc[c] = acc[c] + buf[b, pl.ds(off + 16 * c, 16)]
            for r in range(ch_rows):
                for c in range(4):
                    sem_v[k * ch_rows + r, pl.ds(16 * c, 16)] = (
                        accs[r][c] * scale)

        start(0, buf0, dsem0)

        def loop(j, carry):
            k0 = 2 * j
            k1 = 2 * j + 1
            start(k1, buf1, dsem1)
            wait(buf0, dsem0)
            reduce(buf0, k0)
            start(jnp.minimum(k0 + 2, n_chunks_w - 1), buf0, dsem0)
            wait(buf1, dsem1)
            reduce(buf1, k1)
            return carry

        lax.fori_loop(0, n_chunks_w // 2, loop, 0)
        wait(buf0, dsem0)

        pltpu.sync_copy(sem_v, out_hbm.at[pl.ds(wid * ROWS_W, ROWS_W)])

    return _gather_pool


_ent_sc = _make_gather_pool(E_CHUNKS_W, E_CH, E_GROUPS, N_ENT, E_CH_ROWS,
                            E_SCALE)
_pred_sc = _make_gather_pool(P_CHUNKS_W, P_CH, P_GROUPS, N_PRED, P_CH_ROWS,
                             P_SCALE)


def _mlp_tc_body(seme_ref, semp_ref, mot_ref, w1_ref, b1_ref, w2_ref, b2_ref,
                 wft_ref, wfb_ref, bf_ref, out_ref):
    x = mot_ref[...]
    h = jnp.maximum(
        jnp.dot(x, w1_ref[...], preferred_element_type=jnp.float32)
        + b1_ref[...], 0.0)
    m = jnp.maximum(
        jnp.dot(h, w2_ref[...], preferred_element_type=jnp.float32)
        + b2_ref[...], 0.0)
    s = seme_ref[...] + semp_ref[...]
    f = (jnp.dot(s, wft_ref[...], preferred_element_type=jnp.float32)
         + jnp.dot(m, wfb_ref[...], preferred_element_type=jnp.float32)
         + bf_ref[...])
    out_ref[...] = jnp.maximum(f, 0.0)


_BM = 1024

_mlp_tc = pl.pallas_call(
    _mlp_tc_body,
    grid=(BATCH // _BM,),
    in_specs=[
        pl.BlockSpec((_BM, D), lambda i: (i, 0)),
        pl.BlockSpec((_BM, D), lambda i: (i, 0)),
        pl.BlockSpec((_BM, D), lambda i: (i, 0)),
        pl.BlockSpec((D, D), lambda i: (0, 0)),
        pl.BlockSpec((1, D), lambda i: (0, 0)),
        pl.BlockSpec((D, D), lambda i: (0, 0)),
        pl.BlockSpec((1, D), lambda i: (0, 0)),
        pl.BlockSpec((D, OUT_D), lambda i: (0, 0)),
        pl.BlockSpec((D, OUT_D), lambda i: (0, 0)),
        pl.BlockSpec((1, OUT_D), lambda i: (0, 0)),
    ],
    out_specs=pl.BlockSpec((_BM, OUT_D), lambda i: (i, 0)),
    out_shape=jax.ShapeDtypeStruct((BATCH, OUT_D), jnp.float32),
)


def kernel(entity_indices, predicate_indices, motif_counts_batch,
           entity_table, predicate_table, W1, b1, W2, b2, Wf, bf):
    eidx = entity_indices.astype(jnp.int32).reshape(
        BATCH * N_ENT // E_CH_REAL, E_CH_REAL)
    eidx = jnp.pad(eidx, ((0, 0), (0, E_CH - E_CH_REAL)), mode="edge")
    pidx = predicate_indices.astype(jnp.int32).reshape(
        BATCH * N_PRED // P_CH, P_CH)
    e_hi = eidx >= E_HALF
    p_hi = pidx >= P_HALF
    eoff = e_hi.astype(jnp.int32) * D
    poff = p_hi.astype(jnp.int32) * D
    eidx2 = eidx - e_hi.astype(jnp.int32) * E_HALF
    pidx2 = pidx - p_hi.astype(jnp.int32) * P_HALF
    ptab = _relayout_p(predicate_table.astype(jnp.float32).T,
                       predicate_table.astype(jnp.float32).T)
    sem_p = _pred_sc(pidx2, poff, ptab)
    etab = _relayout_e(entity_table.astype(jnp.float32).T,
                       entity_table.astype(jnp.float32).T)
    sem_e = _ent_sc(eidx2, eoff, etab)
    return _mlp_tc(sem_e, sem_p, motif_counts_batch,
                   W1, b1.reshape(1, D), W2, b2.reshape(1, D),
                   Wf[:D], Wf[D:], bf.reshape(1, OUT_D))
